# Initial kernel scaffold; baseline (speedup 1.0000x reference)
import jax
import jax.numpy as jnp
from jax.experimental import pallas as pl


def kernel(events_list, W1, b1, W2, b2, W3, b3):
    raise NotImplementedError("write your pallas kernel here")


# speedup vs baseline: 3.8856x; 3.8856x over previous
import jax
import jax.numpy as jnp
from jax import lax
from jax.experimental import pallas as pl
from jax.experimental.pallas import tpu as pltpu
from jax.experimental.pallas import tpu_sc as plsc

_C, _H, _W = 9, 180, 240
_WH = _W * _H
_HP = 128
_EV_CHUNK = 8000

_ROWW = 125
_RCH = 72
_ZB = 6480


def _tc_body(ev_ref, w1_ref, b1_ref, w2t_ref, b2_ref, w3_ref, b3_ref,
             idx_ref, val_ref, *, blocks_per_batch, num_voxels, half):
    blk = pl.program_id(0)
    batch = blk // blocks_per_batch
    ev = ev_ref[...]
    x = ev[:, 0:1]
    y = ev[:, 1:2]
    t = ev[:, 2:3]
    p = ev[:, 3:4]
    p2 = (p + 1.0) * 0.5
    base = x + 240.0 * y
    base = base + 388800.0 * p2
    base = base + 777600.0 * batch.astype(jnp.float32)
    offs = (jnp.arange(_C, dtype=jnp.int32).astype(jnp.float32)
            * float(_WH))[None, :]
    idxf = jnp.clip(base + offs, 0.0, float(num_voxels - 1))
    local_off = (batch // 2) * half
    idx_ref[...] = idxf.astype(jnp.int32) - local_off

    w1 = w1_ref[...]
    b1 = b1_ref[...]
    w2t = w2t_ref[...]
    b2 = b2_ref[...]
    w3 = w3_ref[...]
    b3 = b3_ref[0, 0]
    for i in range(_C):
        u = t - (i / (_C - 1))
        h1 = u * w1 + b1
        h1 = jnp.where(h1 >= 0.0, h1, 0.1 * h1)
        h2 = jnp.dot(h1, w2t, preferred_element_type=jnp.float32) + b2
        h2 = jnp.where(h2 >= 0.0, h2, 0.1 * h2)
        o = jnp.sum(h2 * w3, axis=1, keepdims=True) + b3
        val_ref[:, i:i + 1] = t * o


def _tc_values_and_indices(ev4, w1p, b1p, w2tp, b2p, w3p, b3p, n_per_batch,
                           num_voxels, half):
    nev = ev4.shape[0]
    chunk = _EV_CHUNK
    nblk = nev // chunk
    blocks_per_batch = n_per_batch // chunk
    import functools
    body = functools.partial(_tc_body, blocks_per_batch=blocks_per_batch,
                             num_voxels=num_voxels, half=half)
    return pl.pallas_call(
        body,
        grid=(nblk,),
        in_specs=[
            pl.BlockSpec((chunk, 4), lambda i: (i, 0)),
            pl.BlockSpec((1, _HP), lambda i: (0, 0)),
            pl.BlockSpec((1, _HP), lambda i: (0, 0)),
            pl.BlockSpec((_HP, _HP), lambda i: (0, 0)),
            pl.BlockSpec((1, _HP), lambda i: (0, 0)),
            pl.BlockSpec((1, _HP), lambda i: (0, 0)),
            pl.BlockSpec((1, 1), lambda i: (0, 0)),
        ],
        out_specs=[
            pl.BlockSpec((chunk, _C), lambda i: (i, 0)),
            pl.BlockSpec((chunk, _C), lambda i: (i, 0)),
        ],
        out_shape=[
            jax.ShapeDtypeStruct((nev, _C), jnp.int32),
            jax.ShapeDtypeStruct((nev, _C), jnp.float32),
        ],
    )(ev4, w1p, b1p, w2tp, b2p, w3p, b3p)


def _sc_scatter(idx2, val2, num_voxels, half):
    nrows = idx2.shape[0]
    rows_per_tile = nrows // 32
    n_chunks = rows_per_tile // _RCH
    gwords = half // 16
    mesh = plsc.VectorSubcoreMesh(core_axis_name="c", subcore_axis_name="s")

    def body(idx_hbm, val_hbm, out_hbm, idx_v, val_v, zb_v, grid_s):
        cid = lax.axis_index("c")
        sid = lax.axis_index("s")

        def zf(i, carry):
            zb_v[pl.ds(i * 16, 16)] = jnp.zeros((16,), jnp.float32)
            return carry
        lax.fori_loop(0, _ZB // 16, zf, 0)
        gbase = sid * gwords

        def zc(j, carry):
            pltpu.sync_copy(zb_v, grid_s.at[pl.ds(gbase + j * _ZB, _ZB)])
            return carry
        lax.fori_loop(0, gwords // _ZB, zc, 0)
        plsc.subcore_barrier()

        rbase = (cid * 16 + sid) * rows_per_tile

        def sc_chunk(r, carry):
            row0 = rbase + r * _RCH
            pltpu.sync_copy(idx_hbm.at[pl.ds(row0, _RCH)], idx_v)
            pltpu.sync_copy(val_hbm.at[pl.ds(row0, _RCH)], val_v)

            def scat(j, c2):
                pltpu.sync_copy(val_v.at[j], grid_s.at[idx_v.at[j]], add=True)
                return c2
            lax.fori_loop(0, _RCH, scat, 0)
            return carry
        lax.fori_loop(0, n_chunks, sc_chunk, 0)
        plsc.subcore_barrier()

        def oc(j, carry):
            pltpu.sync_copy(grid_s.at[pl.ds(gbase + j * _ZB, _ZB)], zb_v)
            pltpu.sync_copy(zb_v,
                            out_hbm.at[pl.ds(cid * half + gbase + j * _ZB,
                                             _ZB)])
            return carry
        lax.fori_loop(0, gwords // _ZB, oc, 0)

    f = pl.kernel(
        body,
        out_type=jax.ShapeDtypeStruct((num_voxels,), jnp.float32),
        mesh=mesh,
        scratch_types=[
            pltpu.VMEM((_RCH, _ROWW), jnp.int32),
            pltpu.VMEM((_RCH, _ROWW), jnp.float32),
            pltpu.VMEM((_ZB,), jnp.float32),
            pltpu.VMEM_SHARED((half,), jnp.float32),
        ],
    )
    return f(idx2, val2)


def kernel(events_list, W1, b1, W2, b2, W3, b3):
    Bn, Nn = events_list.shape[0], events_list.shape[1]
    num_voxels = 2 * _C * _H * _W * Bn
    half = num_voxels // 2

    ev4 = events_list.reshape(Bn * Nn, 4)
    w1p = jnp.zeros((1, _HP), jnp.float32).at[0, :100].set(W1[:, 0])
    b1p = jnp.zeros((1, _HP), jnp.float32).at[0, :100].set(b1)
    w2tp = jnp.zeros((_HP, _HP), jnp.float32).at[:100, :100].set(W2.T)
    b2p = jnp.zeros((1, _HP), jnp.float32).at[0, :100].set(b2)
    w3p = jnp.zeros((1, _HP), jnp.float32).at[0, :100].set(W3[0, :])
    b3p = b3.reshape(1, 1)

    idx, val = _tc_values_and_indices(ev4, w1p, b1p, w2tp, b2p, w3p, b3p,
                                      Nn, num_voxels, half)
    npairs = Bn * Nn * _C
    idx2 = idx.reshape(npairs // _ROWW, _ROWW)
    val2 = val.reshape(npairs // _ROWW, _ROWW)
    vox = _sc_scatter(idx2, val2, num_voxels, half)
    return vox.reshape(Bn, 2 * _C, _H, _W)


# speedup vs baseline: 3.9043x; 1.0048x over previous
import jax
import jax.numpy as jnp
from jax import lax
from jax.experimental import pallas as pl
from jax.experimental.pallas import tpu as pltpu
from jax.experimental.pallas import tpu_sc as plsc

_C, _H, _W = 9, 180, 240
_WH = _W * _H
_HP = 128
_EV_CHUNK = 8000

_PCH = 9000
_ZB = 6480


def _tc_body(ev_ref, w1_ref, b1_ref, w2t_ref, b2_ref, w3_ref, b3_ref,
             idx_ref, val_ref, *, blocks_per_batch, num_voxels, half):
    blk = pl.program_id(0)
    batch = blk // blocks_per_batch
    ev = ev_ref[...]
    x = ev[:, 0:1]
    y = ev[:, 1:2]
    t = ev[:, 2:3]
    p = ev[:, 3:4]
    p2 = (p + 1.0) * 0.5
    base = x + 240.0 * y
    base = base + 388800.0 * p2
    base = base + 777600.0 * batch.astype(jnp.float32)
    offs = (jnp.arange(_C, dtype=jnp.int32).astype(jnp.float32)
            * float(_WH))[None, :]
    idxf = jnp.clip(base + offs, 0.0, float(num_voxels - 1))
    local_off = (batch // 2) * half
    idx_ref[...] = idxf.astype(jnp.int32) - local_off

    w1 = w1_ref[...]
    b1 = b1_ref[...]
    w2t = w2t_ref[...]
    b2 = b2_ref[...]
    w3 = w3_ref[...]
    b3 = b3_ref[0, 0]
    for i in range(_C):
        u = t - (i / (_C - 1))
        h1 = u * w1 + b1
        h1 = jnp.where(h1 >= 0.0, h1, 0.1 * h1)
        h2 = jnp.dot(h1, w2t, preferred_element_type=jnp.float32) + b2
        h2 = jnp.where(h2 >= 0.0, h2, 0.1 * h2)
        o = jnp.sum(h2 * w3, axis=1, keepdims=True) + b3
        val_ref[:, i:i + 1] = t * o


def _tc_values_and_indices(ev4, w1p, b1p, w2tp, b2p, w3p, b3p, n_per_batch,
                           num_voxels, half):
    nev = ev4.shape[0]
    chunk = _EV_CHUNK
    nblk = nev // chunk
    blocks_per_batch = n_per_batch // chunk
    import functools
    body = functools.partial(_tc_body, blocks_per_batch=blocks_per_batch,
                             num_voxels=num_voxels, half=half)
    return pl.pallas_call(
        body,
        grid=(nblk,),
        in_specs=[
            pl.BlockSpec((chunk, 4), lambda i: (i, 0)),
            pl.BlockSpec((1, _HP), lambda i: (0, 0)),
            pl.BlockSpec((1, _HP), lambda i: (0, 0)),
            pl.BlockSpec((_HP, _HP), lambda i: (0, 0)),
            pl.BlockSpec((1, _HP), lambda i: (0, 0)),
            pl.BlockSpec((1, _HP), lambda i: (0, 0)),
            pl.BlockSpec((1, 1), lambda i: (0, 0)),
        ],
        out_specs=[
            pl.BlockSpec((chunk, _C), lambda i: (i, 0)),
            pl.BlockSpec((chunk, _C), lambda i: (i, 0)),
        ],
        out_shape=[
            jax.ShapeDtypeStruct((nev, _C), jnp.int32),
            jax.ShapeDtypeStruct((nev, _C), jnp.float32),
        ],
    )(ev4, w1p, b1p, w2tp, b2p, w3p, b3p)


def _sc_scatter(idx2, val2, num_voxels, half):
    npairs = idx2.shape[0]
    pairs_per_tile = npairs // 32
    n_chunks = pairs_per_tile // _PCH
    gwords = half // 16
    mesh = plsc.VectorSubcoreMesh(core_axis_name="c", subcore_axis_name="s")

    def body(idx_hbm, val_hbm, out_hbm, idx_v, val_v, zb_v, grid_s):
        cid = lax.axis_index("c")
        sid = lax.axis_index("s")

        def zf(i, carry):
            zb_v[pl.ds(i * 16, 16)] = jnp.zeros((16,), jnp.float32)
            return carry
        lax.fori_loop(0, _ZB // 16, zf, 0)
        gbase = sid * gwords

        def zc(j, carry):
            pltpu.sync_copy(zb_v, grid_s.at[pl.ds(gbase + j * _ZB, _ZB)])
            return carry
        lax.fori_loop(0, gwords // _ZB, zc, 0)
        plsc.subcore_barrier()

        rbase = (cid * 16 + sid) * pairs_per_tile

        def sc_chunk(r, carry):
            p0 = rbase + r * _PCH
            pltpu.sync_copy(idx_hbm.at[pl.ds(p0, _PCH)], idx_v)
            pltpu.sync_copy(val_hbm.at[pl.ds(p0, _PCH)], val_v)
            pltpu.sync_copy(val_v, grid_s.at[idx_v], add=True)
            return carry
        lax.fori_loop(0, n_chunks, sc_chunk, 0)
        plsc.subcore_barrier()

        def oc(j, carry):
            pltpu.sync_copy(grid_s.at[pl.ds(gbase + j * _ZB, _ZB)], zb_v)
            pltpu.sync_copy(zb_v,
                            out_hbm.at[pl.ds(cid * half + gbase + j * _ZB,
                                             _ZB)])
            return carry
        lax.fori_loop(0, gwords // _ZB, oc, 0)

    f = pl.kernel(
        body,
        out_type=jax.ShapeDtypeStruct((num_voxels,), jnp.float32),
        mesh=mesh,
        scratch_types=[
            pltpu.VMEM((_PCH,), jnp.int32),
            pltpu.VMEM((_PCH,), jnp.float32),
            pltpu.VMEM((_ZB,), jnp.float32),
            pltpu.VMEM_SHARED((half,), jnp.float32),
        ],
    )
    return f(idx2, val2)


def kernel(events_list, W1, b1, W2, b2, W3, b3):
    Bn, Nn = events_list.shape[0], events_list.shape[1]
    num_voxels = 2 * _C * _H * _W * Bn
    half = num_voxels // 2

    ev4 = events_list.reshape(Bn * Nn, 4)
    w1p = jnp.zeros((1, _HP), jnp.float32).at[0, :100].set(W1[:, 0])
    b1p = jnp.zeros((1, _HP), jnp.float32).at[0, :100].set(b1)
    w2tp = jnp.zeros((_HP, _HP), jnp.float32).at[:100, :100].set(W2.T)
    b2p = jnp.zeros((1, _HP), jnp.float32).at[0, :100].set(b2)
    w3p = jnp.zeros((1, _HP), jnp.float32).at[0, :100].set(W3[0, :])
    b3p = b3.reshape(1, 1)

    idx, val = _tc_values_and_indices(ev4, w1p, b1p, w2tp, b2p, w3p, b3p,
                                      Nn, num_voxels, half)
    idx2 = idx.reshape(-1)
    val2 = val.reshape(-1)
    vox = _sc_scatter(idx2, val2, num_voxels, half)
    return vox.reshape(Bn, 2 * _C, _H, _W)


# speedup vs baseline: 8.2581x; 2.1151x over previous
import jax
import jax.numpy as jnp
from jax import lax
from jax.experimental import pallas as pl
from jax.experimental.pallas import tpu as pltpu
from jax.experimental.pallas import tpu_sc as plsc

_C, _H, _W = 9, 180, 240
_WH = _W * _H
_HP = 128
_EV_CHUNK = 8192

_EC = 5000
_ZB = 6480


def _tc_body(t_ref, w1_ref, b1_ref, w2_ref, b2_ref, w3_ref, b3_ref, *o_refs):
    t = t_ref[...][None, :]
    w1 = w1_ref[...]
    b1 = b1_ref[...]
    w2 = w2_ref[...]
    b2 = b2_ref[...]
    w3 = w3_ref[...]
    b3 = b3_ref[0, 0]
    for i in range(_C):
        u = t - (i / (_C - 1))
        h1 = w1 * u + b1
        h1 = jnp.where(h1 >= 0.0, h1, 0.1 * h1)
        h2 = jnp.dot(w2, h1, preferred_element_type=jnp.float32) + b2
        h2 = jnp.where(h2 >= 0.0, h2, 0.1 * h2)
        o = jnp.sum(h2 * w3, axis=0, keepdims=True) + b3
        o_refs[i][...] = (t * o)[0]


def _tc_values(t1d, w1p, b1p, w2p, b2p, w3p, b3p):
    nev = t1d.shape[0]
    chunk = _EV_CHUNK
    nblk = pl.cdiv(nev, chunk)
    return pl.pallas_call(
        _tc_body,
        grid=(nblk,),
        in_specs=[
            pl.BlockSpec((chunk,), lambda i: (i,)),
            pl.BlockSpec((_HP, 1), lambda i: (0, 0)),
            pl.BlockSpec((_HP, 1), lambda i: (0, 0)),
            pl.BlockSpec((_HP, _HP), lambda i: (0, 0)),
            pl.BlockSpec((_HP, 1), lambda i: (0, 0)),
            pl.BlockSpec((_HP, 1), lambda i: (0, 0)),
            pl.BlockSpec((1, 1), lambda i: (0, 0)),
        ],
        out_specs=[pl.BlockSpec((chunk,), lambda i: (i,))
                   for _ in range(_C)],
        out_shape=[jax.ShapeDtypeStruct((nev,), jnp.float32)
                   for _ in range(_C)],
    )(t1d, w1p, b1p, w2p, b2p, w3p, b3p)


def _sc_scatter(x1d, y1d, p1d, vals, num_voxels, half):
    nev = x1d.shape[0]
    ev_per_core = nev // 2
    ev_per_tile = ev_per_core // 16
    n_chunks = ev_per_tile // _EC
    nfull = _EC // 16
    has_tail = (_EC % 16) != 0
    gwords = half // 16
    mesh = plsc.VectorSubcoreMesh(core_axis_name="c", subcore_axis_name="s")

    def body(x_hbm, y_hbm, p_hbm, *rest):
        v_hbms = rest[:_C]
        out_hbm = rest[_C]
        (x_v, y_v, p_v, idx_v, val_v, zb_v, grid_s) = rest[_C + 1:]
        cid = lax.axis_index("c")
        sid = lax.axis_index("s")

        def zf(i, carry):
            zb_v[pl.ds(i * 16, 16)] = jnp.zeros((16,), jnp.float32)
            return carry
        lax.fori_loop(0, _ZB // 16, zf, 0)
        gbase = sid * gwords

        def zc(j, carry):
            pltpu.sync_copy(zb_v, grid_s.at[pl.ds(gbase + j * _ZB, _ZB)])
            return carry
        lax.fori_loop(0, gwords // _ZB, zc, 0)
        plsc.subcore_barrier()

        ebase = cid * ev_per_core + sid * ev_per_tile
        batch = cid * 2 + sid // 8
        bterm = batch.astype(jnp.float32) * 777600.0
        loc = cid * half

        def base_of(s):
            xx = x_v[pl.ds(s, 16)]
            yy = y_v[pl.ds(s, 16)]
            pp = p_v[pl.ds(s, 16)]
            p2 = (pp + 1.0) * 0.5
            b = xx + 240.0 * yy
            b = b + 388800.0 * p2
            return b + bterm

        def to_idx(f, off):
            f = jnp.clip(f + off, 0.0, float(num_voxels - 1))
            return f.astype(jnp.int32) - loc

        def sc_chunk(r, carry):
            e0 = ebase + r * _EC
            pltpu.sync_copy(x_hbm.at[pl.ds(e0, _EC)], x_v)
            pltpu.sync_copy(y_hbm.at[pl.ds(e0, _EC)], y_v)
            pltpu.sync_copy(p_hbm.at[pl.ds(e0, _EC)], p_v)

            if has_tail:
                zb_v[pl.ds(0, 16)] = base_of(_EC - 16)

            def bvec(v, carry2):
                s = v * 16
                x_v[pl.ds(s, 16)] = base_of(s)
                return carry2
            lax.fori_loop(0, nfull, bvec, 0)

            for i in range(_C):
                off = float(_WH * i)

                def ivec(v, carry3):
                    s = v * 16
                    idx_v[pl.ds(s, 16)] = to_idx(x_v[pl.ds(s, 16)], off)
                    return carry3
                lax.fori_loop(0, nfull, ivec, 0)
                if has_tail:
                    idx_v[pl.ds(_EC - 16, 16)] = to_idx(zb_v[pl.ds(0, 16)],
                                                        off)
                pltpu.sync_copy(v_hbms[i].at[pl.ds(e0, _EC)], val_v)
                pltpu.sync_copy(val_v, grid_s.at[idx_v], add=True)
            return carry
        lax.fori_loop(0, n_chunks, sc_chunk, 0)
        plsc.subcore_barrier()

        def oc(j, carry):
            pltpu.sync_copy(grid_s.at[pl.ds(gbase + j * _ZB, _ZB)], zb_v)
            pltpu.sync_copy(zb_v,
                            out_hbm.at[pl.ds(cid * half + gbase + j * _ZB,
                                             _ZB)])
            return carry
        lax.fori_loop(0, gwords // _ZB, oc, 0)

    f = pl.kernel(
        body,
        out_type=jax.ShapeDtypeStruct((num_voxels,), jnp.float32),
        mesh=mesh,
        scratch_types=[
            pltpu.VMEM((_EC,), jnp.float32),
            pltpu.VMEM((_EC,), jnp.float32),
            pltpu.VMEM((_EC,), jnp.float32),
            pltpu.VMEM((_EC,), jnp.int32),
            pltpu.VMEM((_EC,), jnp.float32),
            pltpu.VMEM((_ZB,), jnp.float32),
            pltpu.VMEM_SHARED((half,), jnp.float32),
        ],
    )
    return f(x1d, y1d, p1d, *vals)


def kernel(events_list, W1, b1, W2, b2, W3, b3):
    Bn, Nn = events_list.shape[0], events_list.shape[1]
    num_voxels = 2 * _C * _H * _W * Bn
    half = num_voxels // 2

    x1d = events_list[:, :, 0].reshape(-1)
    y1d = events_list[:, :, 1].reshape(-1)
    t1d = events_list[:, :, 2].reshape(-1)
    p1d = events_list[:, :, 3].reshape(-1)
    w1p = jnp.zeros((_HP, 1), jnp.float32).at[:100, 0].set(W1[:, 0])
    b1p = jnp.zeros((_HP, 1), jnp.float32).at[:100, 0].set(b1)
    w2p = jnp.zeros((_HP, _HP), jnp.float32).at[:100, :100].set(W2)
    b2p = jnp.zeros((_HP, 1), jnp.float32).at[:100, 0].set(b2)
    w3p = jnp.zeros((_HP, 1), jnp.float32).at[:100, 0].set(W3[0, :])
    b3p = b3.reshape(1, 1)

    vals = _tc_values(t1d, w1p, b1p, w2p, b2p, w3p, b3p)
    vox = _sc_scatter(x1d, y1d, p1d, vals, num_voxels, half)
    return vox.reshape(Bn, 2 * _C, _H, _W)


# speedup vs baseline: 10.7405x; 1.3006x over previous
import jax
import jax.numpy as jnp
from jax import lax
from jax.experimental import pallas as pl
from jax.experimental.pallas import tpu as pltpu
from jax.experimental.pallas import tpu_sc as plsc

_C, _H, _W = 9, 180, 240
_WH = _W * _H
_HP = 128
_EV_CHUNK = 8192

_EC = 5000
_ZB = 6480


def _tc_body(t_ref, w1_ref, b1s_ref, w2_ref, b2_ref, w3_ref, b3_ref,
             *o_refs):
    t = t_ref[...][None, :]
    w1 = w1_ref[...]
    w2 = w2_ref[...]
    b2 = b2_ref[...]
    w3 = w3_ref[...]
    b3 = b3_ref[0, 0]
    g = w1 * t
    for i in range(_C):
        h1 = g + b1s_ref[:, i:i + 1]
        h1 = jnp.maximum(h1, 0.1 * h1)
        h2 = jnp.dot(w2, h1, preferred_element_type=jnp.float32) + b2
        h2 = jnp.maximum(h2, 0.1 * h2)
        o = jnp.sum(h2 * w3, axis=0, keepdims=True) + b3
        o_refs[i][...] = (t * o)[0]


def _tc_values(t1d, w1p, b1p, w2p, b2p, w3p, b3p):
    nev = t1d.shape[0]
    chunk = _EV_CHUNK
    nblk = pl.cdiv(nev, chunk)
    return pl.pallas_call(
        _tc_body,
        grid=(nblk,),
        in_specs=[
            pl.BlockSpec((chunk,), lambda i: (i,)),
            pl.BlockSpec((_HP, 1), lambda i: (0, 0)),
            pl.BlockSpec((_HP, _C), lambda i: (0, 0)),
            pl.BlockSpec((_HP, _HP), lambda i: (0, 0)),
            pl.BlockSpec((_HP, 1), lambda i: (0, 0)),
            pl.BlockSpec((_HP, 1), lambda i: (0, 0)),
            pl.BlockSpec((1, 1), lambda i: (0, 0)),
        ],
        out_specs=[pl.BlockSpec((chunk,), lambda i: (i,))
                   for _ in range(_C)],
        out_shape=[jax.ShapeDtypeStruct((nev,), jnp.float32)
                   for _ in range(_C)],
    )(t1d, w1p, b1p, w2p, b2p, w3p, b3p)


def _sc_scatter(x1d, y1d, p1d, vals, num_voxels, half):
    nev = x1d.shape[0]
    ev_per_core = nev // 2
    ev_per_tile = ev_per_core // 16
    n_chunks = ev_per_tile // _EC
    nfull = _EC // 16
    has_tail = (_EC % 16) != 0
    gwords = half // 16
    mesh = plsc.VectorSubcoreMesh(core_axis_name="c", subcore_axis_name="s")

    def body(x_hbm, y_hbm, p_hbm, *rest):
        v_hbms = rest[:_C]
        out_hbm = rest[_C]
        (x_v, y_v, p_v, idx_v, val_v, zb_v, grid_s) = rest[_C + 1:]
        cid = lax.axis_index("c")
        sid = lax.axis_index("s")

        def zf(i, carry):
            zb_v[pl.ds(i * 16, 16)] = jnp.zeros((16,), jnp.float32)
            return carry
        lax.fori_loop(0, _ZB // 16, zf, 0)
        gbase = sid * gwords

        def zc(j, carry):
            pltpu.sync_copy(zb_v, grid_s.at[pl.ds(gbase + j * _ZB, _ZB)])
            return carry
        lax.fori_loop(0, gwords // _ZB, zc, 0)
        plsc.subcore_barrier()

        ebase = cid * ev_per_core + sid * ev_per_tile
        batch = cid * 2 + sid // 8
        bterm = batch.astype(jnp.float32) * 777600.0
        loc = cid * half

        def base_of(s):
            xx = x_v[pl.ds(s, 16)]
            yy = y_v[pl.ds(s, 16)]
            pp = p_v[pl.ds(s, 16)]
            p2 = (pp + 1.0) * 0.5
            b = xx + 240.0 * yy
            b = b + 388800.0 * p2
            return b + bterm

        def to_idx(f, off):
            f = jnp.clip(f + off, 0.0, float(num_voxels - 1))
            return f.astype(jnp.int32) - loc

        def sc_chunk(r, carry):
            e0 = ebase + r * _EC
            pltpu.sync_copy(x_hbm.at[pl.ds(e0, _EC)], x_v)
            pltpu.sync_copy(y_hbm.at[pl.ds(e0, _EC)], y_v)
            pltpu.sync_copy(p_hbm.at[pl.ds(e0, _EC)], p_v)

            if has_tail:
                zb_v[pl.ds(0, 16)] = base_of(_EC - 16)

            def bvec(v, carry2):
                s = v * 16
                x_v[pl.ds(s, 16)] = base_of(s)
                return carry2
            lax.fori_loop(0, nfull, bvec, 0)

            for i in range(_C):
                off = float(_WH * i)

                def ivec(v, carry3):
                    s = v * 16
                    idx_v[pl.ds(s, 16)] = to_idx(x_v[pl.ds(s, 16)], off)
                    return carry3
                lax.fori_loop(0, nfull, ivec, 0)
                if has_tail:
                    idx_v[pl.ds(_EC - 16, 16)] = to_idx(zb_v[pl.ds(0, 16)],
                                                        off)
                pltpu.sync_copy(v_hbms[i].at[pl.ds(e0, _EC)], val_v)
                pltpu.sync_copy(val_v, grid_s.at[idx_v], add=True)
            return carry
        lax.fori_loop(0, n_chunks, sc_chunk, 0)
        plsc.subcore_barrier()

        def oc(j, carry):
            pltpu.sync_copy(grid_s.at[pl.ds(gbase + j * _ZB, _ZB)], zb_v)
            pltpu.sync_copy(zb_v,
                            out_hbm.at[pl.ds(cid * half + gbase + j * _ZB,
                                             _ZB)])
            return carry
        lax.fori_loop(0, gwords // _ZB, oc, 0)

    f = pl.kernel(
        body,
        out_type=jax.ShapeDtypeStruct((num_voxels,), jnp.float32),
        mesh=mesh,
        scratch_types=[
            pltpu.VMEM((_EC,), jnp.float32),
            pltpu.VMEM((_EC,), jnp.float32),
            pltpu.VMEM((_EC,), jnp.float32),
            pltpu.VMEM((_EC,), jnp.int32),
            pltpu.VMEM((_EC,), jnp.float32),
            pltpu.VMEM((_ZB,), jnp.float32),
            pltpu.VMEM_SHARED((half,), jnp.float32),
        ],
    )
    return f(x1d, y1d, p1d, *vals)


def kernel(events_list, W1, b1, W2, b2, W3, b3):
    Bn, Nn = events_list.shape[0], events_list.shape[1]
    num_voxels = 2 * _C * _H * _W * Bn
    half = num_voxels // 2

    x1d = events_list[:, :, 0].reshape(-1)
    y1d = events_list[:, :, 1].reshape(-1)
    t1d = events_list[:, :, 2].reshape(-1)
    p1d = events_list[:, :, 3].reshape(-1)
    w1p = jnp.zeros((_HP, 1), jnp.float32).at[:100, 0].set(W1[:, 0])
    cs = (jnp.arange(_C, dtype=jnp.float32) / (_C - 1))[None, :]
    b1p = (jnp.zeros((_HP, _C), jnp.float32)
           .at[:100, :].set(b1[:, None] - W1[:, 0:1] * cs))
    w2p = jnp.zeros((_HP, _HP), jnp.float32).at[:100, :100].set(W2)
    b2p = jnp.zeros((_HP, 1), jnp.float32).at[:100, 0].set(b2)
    w3p = jnp.zeros((_HP, 1), jnp.float32).at[:100, 0].set(W3[0, :])
    b3p = b3.reshape(1, 1)

    vals = _tc_values(t1d, w1p, b1p, w2p, b2p, w3p, b3p)
    vox = _sc_scatter(x1d, y1d, p1d, vals, num_voxels, half)
    return vox.reshape(Bn, 2 * _C, _H, _W)
